# baseline (speedup 1.0000x reference)
import functools

import jax
import jax.numpy as jnp
from jax import lax
from jax.experimental import pallas as pl
from jax.experimental.pallas import tpu as pltpu
from jax.experimental.pallas import tpu_sc as plsc

VOCAB = 1000000
SEQ_LEN = 200
DIM = 32
HALF = 16

NUM_CORES = 2
NUM_SUBCORES = 16
NUM_WORKERS = NUM_CORES * NUM_SUBCORES

BATCH = 4096
ROWS_PER_WORKER = BATCH // NUM_WORKERS

NBUF = 4
CHUNK_BROWS = 2
N_CHUNKS = ROWS_PER_WORKER // CHUNK_BROWS
N_ITERS = N_CHUNKS // NBUF
IDX_PER_DMA = 100
DMAS_PER_BROW = SEQ_LEN // IDX_PER_DMA
IDX_GROUPS = N_CHUNKS * CHUNK_BROWS * DMAS_PER_BROW

_mesh = plsc.VectorSubcoreMesh(core_axis_name="c", subcore_axis_name="s")


def _eye():
    r = lax.broadcasted_iota(jnp.int32, (DIM, DIM), 0)
    c = lax.broadcasted_iota(jnp.int32, (DIM, DIM), 1)
    return (r == c).astype(jnp.float32)


VCHUNK = 8192
VGRID = -(-VOCAB // VCHUNK)


def _detile_body(wt_ref, out_ref):
    out_ref[...] = lax.dot_general(
        wt_ref[...],
        _eye(),
        (((0,), (0,)), ((), ())),
        precision=lax.Precision.HIGHEST,
        preferred_element_type=jnp.float32,
    )


_detile = pl.pallas_call(
    _detile_body,
    grid=(VGRID,),
    in_specs=[pl.BlockSpec((DIM, VCHUNK), lambda i: (0, i))],
    out_specs=pl.BlockSpec((VCHUNK, DIM), lambda i: (i, 0)),
    out_shape=jax.ShapeDtypeStruct((VOCAB, DIM), jnp.float32),
)


LBLK = 8
LGRID = SEQ_LEN // LBLK


def _transpose_out_body(x_ref, out_ref):
    eye = _eye()
    for j in range(LBLK):
        out_ref[j] = lax.dot_general(
            eye,
            x_ref[:, j, :],
            (((1,), (1,)), ((), ())),
            precision=lax.Precision.HIGHEST,
            preferred_element_type=jnp.float32,
        )


_transpose_out = pl.pallas_call(
    _transpose_out_body,
    grid=(LGRID,),
    in_specs=[pl.BlockSpec((BATCH, LBLK, DIM), lambda i: (0, i, 0))],
    out_specs=pl.BlockSpec((LBLK, DIM, BATCH), lambda i: (i, 0, 0)),
    out_shape=jax.ShapeDtypeStruct((SEQ_LEN, DIM, BATCH), jnp.float32),
)


@functools.partial(
    pl.kernel,
    out_type=jax.ShapeDtypeStruct((BATCH, SEQ_LEN, DIM), jnp.float32),
    mesh=_mesh,
    scratch_types=[
        pltpu.VMEM((IDX_GROUPS, IDX_PER_DMA), jnp.int32),
        pltpu.VMEM((SEQ_LEN, DIM), jnp.float32),
        [pltpu.VMEM((CHUNK_BROWS, SEQ_LEN, DIM), jnp.float32) for _ in range(NBUF)],
        [pltpu.SemaphoreType.DMA for _ in range(NBUF)],
        [pltpu.SemaphoreType.DMA for _ in range(NBUF)],
    ],
    compiler_params=pltpu.CompilerParams(use_tc_tiling_on_sc=False),
)
def _emb_lookup(idx_hbm, pos_hbm, table_hbm, out_hbm, idx_v, pos_v, bufs, gsems, osems):
    wid = lax.axis_index("s") * NUM_CORES + lax.axis_index("c")
    base = wid * ROWS_PER_WORKER

    pltpu.sync_copy(idx_hbm.at[wid], idx_v)
    pltpu.sync_copy(pos_hbm, pos_v)

    def issue_gathers(c, b):
        lb = c * CHUNK_BROWS
        for s in range(CHUNK_BROWS):
            for g in range(DMAS_PER_BROW):
                pltpu.async_copy(
                    table_hbm.at[idx_v.at[(lb + s) * DMAS_PER_BROW + g]],
                    bufs[b].at[s, pl.ds(g * IDX_PER_DMA, IDX_PER_DMA)],
                    gsems[b],
                )

    def wait_gathers(c, b):
        lb = c * CHUNK_BROWS
        for s in range(CHUNK_BROWS):
            for g in range(DMAS_PER_BROW):
                pltpu.make_async_copy(
                    table_hbm.at[idx_v.at[(lb + s) * DMAS_PER_BROW + g]],
                    bufs[b].at[s, pl.ds(g * IDX_PER_DMA, IDX_PER_DMA)],
                    gsems[b],
                ).wait()

    def write_desc(c, b):
        return pltpu.make_async_copy(
            bufs[b],
            out_hbm.at[pl.ds(base + c * CHUNK_BROWS, CHUNK_BROWS)],
            osems[b],
        )

    def add_pos(b):
        def add_body(j, carry):
            pv0 = pos_v[j, pl.ds(0, HALF)]
            pv1 = pos_v[j, pl.ds(HALF, HALF)]
            for s in range(CHUNK_BROWS):
                plsc.addupdate(bufs[b].at[s, j, pl.ds(0, HALF)], pv0)
                plsc.addupdate(bufs[b].at[s, j, pl.ds(HALF, HALF)], pv1)
            return carry

        lax.fori_loop(0, SEQ_LEN, add_body, 0)

    issue_gathers(0, 0)
    issue_gathers(1, 1)

    def iter_body(i, carry):
        c0 = i * NBUF
        for b in range(NBUF):
            c = c0 + b
            wait_gathers(c, b)
            add_pos(b)
            write_desc(c, b).start()
            bp = (b + 2) % NBUF
            cp = c + 2

            def prefetch():
                write_desc(cp - NBUF, bp).wait()
                issue_gathers(cp, bp)

            def first_prefetch():
                issue_gathers(cp, bp)

            if b < 2:
                lax.cond(i > 0, prefetch, first_prefetch)
            else:
                def wait_only():
                    write_desc(cp - NBUF, bp).wait()

                lax.cond(i < N_ITERS - 1, prefetch, wait_only)
        return carry

    lax.fori_loop(0, N_ITERS, iter_body, 0)

    write_desc(N_CHUNKS - 2, 2).wait()
    write_desc(N_CHUNKS - 1, 3).wait()


def kernel(inputs, word_table, pos_table):
    idx = inputs.astype(jnp.int32).reshape(NUM_WORKERS, IDX_GROUPS, IDX_PER_DMA)
    table_rm = _detile(word_table.T)
    out = _emb_lookup(idx, pos_table, table_rm)
    return _transpose_out(out).transpose(2, 0, 1)


# speedup vs baseline: 1.5251x; 1.5251x over previous
import functools

import jax
import jax.numpy as jnp
from jax import lax
from jax.experimental import pallas as pl
from jax.experimental.pallas import tpu as pltpu
from jax.experimental.pallas import tpu_sc as plsc

SEQ_LEN = 200
DIM = 32
HALF = 16

NUM_CORES = 2
NUM_SUBCORES = 16
NUM_WORKERS = NUM_CORES * NUM_SUBCORES

BATCH = 4096
ROWS_PER_WORKER = BATCH // NUM_WORKERS

NBUF = 4
CHUNK_BROWS = 2
N_CHUNKS = ROWS_PER_WORKER // CHUNK_BROWS
N_ITERS = N_CHUNKS // NBUF
IDX_PER_DMA = 100
DMAS_PER_BROW = SEQ_LEN // IDX_PER_DMA
IDX_GROUPS = N_CHUNKS * CHUNK_BROWS * DMAS_PER_BROW

_mesh = plsc.VectorSubcoreMesh(core_axis_name="c", subcore_axis_name="s")


@functools.partial(
    pl.kernel,
    out_type=jax.ShapeDtypeStruct((BATCH, SEQ_LEN, DIM), jnp.float32),
    mesh=_mesh,
    scratch_types=[
        pltpu.VMEM((IDX_GROUPS, IDX_PER_DMA), jnp.int32),
        pltpu.VMEM((SEQ_LEN, DIM), jnp.float32),
        [pltpu.VMEM((CHUNK_BROWS, SEQ_LEN, DIM), jnp.bfloat16) for _ in range(NBUF)],
        [pltpu.VMEM((CHUNK_BROWS, SEQ_LEN, DIM), jnp.float32) for _ in range(NBUF)],
        [pltpu.SemaphoreType.DMA for _ in range(NBUF)],
        [pltpu.SemaphoreType.DMA for _ in range(NBUF)],
    ],
    compiler_params=pltpu.CompilerParams(
        use_tc_tiling_on_sc=False, needs_layout_passes=False
    ),
)
def _emb_lookup(idx_hbm, pos_hbm, table_hbm, out_hbm, idx_v, pos_v, gbufs, fbufs,
                gsems, osems):
    wid = lax.axis_index("s") * NUM_CORES + lax.axis_index("c")
    base = wid * ROWS_PER_WORKER

    pltpu.sync_copy(idx_hbm.at[wid], idx_v)
    pltpu.sync_copy(pos_hbm, pos_v)

    def issue_gathers(c, b):
        lb = c * CHUNK_BROWS
        for s in range(CHUNK_BROWS):
            for g in range(DMAS_PER_BROW):
                pltpu.async_copy(
                    table_hbm.at[idx_v.at[(lb + s) * DMAS_PER_BROW + g]],
                    gbufs[b].at[s, pl.ds(g * IDX_PER_DMA, IDX_PER_DMA)],
                    gsems[b],
                )

    def wait_gathers(c, b):
        lb = c * CHUNK_BROWS
        for s in range(CHUNK_BROWS):
            for g in range(DMAS_PER_BROW):
                pltpu.make_async_copy(
                    table_hbm.at[idx_v.at[(lb + s) * DMAS_PER_BROW + g]],
                    gbufs[b].at[s, pl.ds(g * IDX_PER_DMA, IDX_PER_DMA)],
                    gsems[b],
                ).wait()

    def write_desc(c, b):
        return pltpu.make_async_copy(
            fbufs[b],
            out_hbm.at[pl.ds(base + c * CHUNK_BROWS, CHUNK_BROWS)],
            osems[b],
        )

    def add_pos(b):
        def add_body(j, carry):
            pv0 = pos_v[j, pl.ds(0, HALF)]
            pv1 = pos_v[j, pl.ds(HALF, HALF)]
            for s in range(CHUNK_BROWS):
                row = gbufs[b][s, j, :]
                x, y = plsc.unpack(row, format=plsc.PackFormat.INTERLEAVED)
                fbufs[b][s, j, pl.ds(0, HALF)] = x + pv0
                fbufs[b][s, j, pl.ds(HALF, HALF)] = y + pv1
            return carry

        lax.fori_loop(0, SEQ_LEN, add_body, 0)

    issue_gathers(0, 0)
    issue_gathers(1, 1)

    def iter_body(i, carry):
        c0 = i * NBUF
        for b in range(NBUF):
            c = c0 + b
            wait_gathers(c, b)
            add_pos(b)
            write_desc(c, b).start()
            bp = (b + 2) % NBUF
            cp = c + 2

            def prefetch():
                write_desc(cp - NBUF, bp).wait()
                issue_gathers(cp, bp)

            def first_prefetch():
                issue_gathers(cp, bp)

            if b < 2:
                lax.cond(i > 0, prefetch, first_prefetch)
            else:
                def wait_only():
                    write_desc(cp - NBUF, bp).wait()

                lax.cond(i < N_ITERS - 1, prefetch, wait_only)
        return carry

    lax.fori_loop(0, N_ITERS, iter_body, 0)

    write_desc(N_CHUNKS - 2, 2).wait()
    write_desc(N_CHUNKS - 1, 3).wait()


def kernel(inputs, word_table, pos_table):
    idx = inputs.astype(jnp.int32).reshape(NUM_WORKERS, IDX_GROUPS, IDX_PER_DMA)
    perm = jnp.concatenate([jnp.arange(0, DIM, 2), jnp.arange(1, DIM, 2)])
    eye = lax.optimization_barrier(jnp.eye(DIM, dtype=jnp.float32))
    pmat = eye[perm]
    table16 = (word_table @ eye).astype(jnp.bfloat16)
    out = _emb_lookup(idx, pos_table[:, perm], table16)
    return out @ pmat


# speedup vs baseline: 1.6855x; 1.1051x over previous
import functools

import jax
import jax.numpy as jnp
from jax import lax
from jax.experimental import pallas as pl
from jax.experimental.pallas import tpu as pltpu
from jax.experimental.pallas import tpu_sc as plsc

SEQ_LEN = 200
DIM = 32
HALF = 16

NUM_CORES = 2
NUM_SUBCORES = 16
NUM_WORKERS = NUM_CORES * NUM_SUBCORES

BATCH = 4096
ROWS_PER_WORKER = BATCH // NUM_WORKERS

NBUF = 4
CHUNK_BROWS = 2
N_CHUNKS = ROWS_PER_WORKER // CHUNK_BROWS
N_ITERS = N_CHUNKS // NBUF
IDX_PER_DMA = 100
DMAS_PER_BROW = SEQ_LEN // IDX_PER_DMA
IDX_GROUPS = N_CHUNKS * CHUNK_BROWS * DMAS_PER_BROW

_mesh = plsc.VectorSubcoreMesh(core_axis_name="c", subcore_axis_name="s")


@functools.partial(
    pl.kernel,
    out_type=jax.ShapeDtypeStruct((BATCH, SEQ_LEN, DIM), jnp.float32),
    mesh=_mesh,
    scratch_types=[
        pltpu.VMEM((IDX_GROUPS, IDX_PER_DMA), jnp.int32),
        pltpu.VMEM((SEQ_LEN, DIM), jnp.float32),
        [pltpu.VMEM((CHUNK_BROWS, SEQ_LEN, DIM), jnp.float32) for _ in range(NBUF)],
        [pltpu.SemaphoreType.DMA for _ in range(NBUF)],
        [pltpu.SemaphoreType.DMA for _ in range(NBUF)],
    ],
    compiler_params=pltpu.CompilerParams(use_tc_tiling_on_sc=False),
)
def _emb_lookup(idx_hbm, pos_hbm, table_hbm, out_hbm, idx_v, pos_v, bufs, gsems, osems):
    wid = lax.axis_index("s") * NUM_CORES + lax.axis_index("c")
    base = wid * ROWS_PER_WORKER

    pltpu.sync_copy(idx_hbm.at[wid], idx_v)
    pltpu.sync_copy(pos_hbm, pos_v)

    def issue_gathers(c, b):
        lb = c * CHUNK_BROWS
        for s in range(CHUNK_BROWS):
            for g in range(DMAS_PER_BROW):
                pltpu.async_copy(
                    table_hbm.at[idx_v.at[(lb + s) * DMAS_PER_BROW + g]],
                    bufs[b].at[s, pl.ds(g * IDX_PER_DMA, IDX_PER_DMA)],
                    gsems[b],
                )

    def wait_gathers(c, b):
        lb = c * CHUNK_BROWS
        for s in range(CHUNK_BROWS):
            for g in range(DMAS_PER_BROW):
                pltpu.make_async_copy(
                    table_hbm.at[idx_v.at[(lb + s) * DMAS_PER_BROW + g]],
                    bufs[b].at[s, pl.ds(g * IDX_PER_DMA, IDX_PER_DMA)],
                    gsems[b],
                ).wait()

    def write_desc(c, b):
        return pltpu.make_async_copy(
            bufs[b],
            out_hbm.at[pl.ds(base + c * CHUNK_BROWS, CHUNK_BROWS)],
            osems[b],
        )

    def add_pos(b):
        def add_body(j, carry):
            pv0 = pos_v[j, pl.ds(0, HALF)]
            pv1 = pos_v[j, pl.ds(HALF, HALF)]
            for s in range(CHUNK_BROWS):
                plsc.addupdate(bufs[b].at[s, j, pl.ds(0, HALF)], pv0)
                plsc.addupdate(bufs[b].at[s, j, pl.ds(HALF, HALF)], pv1)
            return carry

        lax.fori_loop(0, SEQ_LEN, add_body, 0)

    issue_gathers(0, 0)
    issue_gathers(1, 1)

    def iter_body(i, carry):
        c0 = i * NBUF
        for b in range(NBUF):
            c = c0 + b
            wait_gathers(c, b)
            add_pos(b)
            write_desc(c, b).start()
            bp = (b + 2) % NBUF
            cp = c + 2

            def prefetch():
                write_desc(cp - NBUF, bp).wait()
                issue_gathers(cp, bp)

            def first_prefetch():
                issue_gathers(cp, bp)

            if b < 2:
                lax.cond(i > 0, prefetch, first_prefetch)
            else:
                def wait_only():
                    write_desc(cp - NBUF, bp).wait()

                lax.cond(i < N_ITERS - 1, prefetch, wait_only)
        return carry

    lax.fori_loop(0, N_ITERS, iter_body, 0)

    write_desc(N_CHUNKS - 2, 2).wait()
    write_desc(N_CHUNKS - 1, 3).wait()


def kernel(inputs, word_table, pos_table):
    idx = inputs.astype(jnp.int32).reshape(NUM_WORKERS, IDX_GROUPS, IDX_PER_DMA)
    eye = lax.optimization_barrier(jnp.eye(DIM, dtype=jnp.float32))
    out = _emb_lookup(idx, pos_table, word_table @ eye)
    return out @ eye
